# baseline (speedup 1.0000x reference)
import functools

import jax
import jax.numpy as jnp
from jax import lax
from jax.experimental import pallas as pl
from jax.experimental.pallas import tpu as pltpu
from jax.experimental.pallas import tpu_sc as plsc

B = 16384
D = 32
_NC = 2
_NS = 16
_NW = _NC * _NS
_BPW = B // _NW
_CHUNK = 128
_NCH = _BPW // _CHUNK
_L = 16
_NG = _BPW // _L

_LN2 = 0.6931471805599453
_SQRT2H = 1.4142135


def _rsqrt(x):
    i = plsc.bitcast(x, jnp.int32)
    i = jnp.int32(0x5F3759DF) - (i >> 1)
    y = plsc.bitcast(i, jnp.float32)
    for _ in range(3):
        y = y * (1.5 - 0.5 * x * y * y)
    return y


def _sqrt(x):
    xs = x + 1e-30
    return xs * _rsqrt(xs)


def _log(w):
    iw = plsc.bitcast(w, jnp.int32)
    e = ((iw >> 23) & jnp.int32(0xFF)) - jnp.int32(127)
    m = plsc.bitcast((iw & jnp.int32(0x007FFFFF)) | jnp.int32(0x3F800000),
                     jnp.float32)
    big = m > _SQRT2H
    m = jnp.where(big, m * 0.5, m)
    e = jnp.where(big, e + 1, e)
    z = (m - 1.0) / (m + 1.0)
    z2 = z * z
    p = 2.0 * z * (1.0 + z2 * (1.0 / 3.0 + z2 * (0.2 + z2 * (1.0 / 7.0))))
    return e.astype(jnp.float32) * _LN2 + p


def _sc_fused(u2, i2, user_emb, item_emb, ub1, ib1, scal):
    mesh = plsc.VectorSubcoreMesh(core_axis_name="c", subcore_axis_name="s")

    @functools.partial(
        pl.kernel,
        mesh=mesh,
        compiler_params=pltpu.CompilerParams(
            use_tc_tiling_on_sc=False, needs_layout_passes=False),
        out_type=jax.ShapeDtypeStruct((B,), jnp.float32),
        scratch_types=[
            pltpu.VMEM((_NCH, _CHUNK), jnp.int32),
            pltpu.VMEM((_NCH, _CHUNK), jnp.int32),
            pltpu.VMEM((_BPW, D), jnp.float32),
            pltpu.VMEM((_BPW, D), jnp.float32),
            pltpu.VMEM((_BPW,), jnp.float32),
            pltpu.VMEM((_BPW,), jnp.float32),
            pltpu.VMEM((_BPW,), jnp.float32),
            pltpu.VMEM((16,), jnp.float32),
            pltpu.SemaphoreType.DMA,
        ],
    )
    def fused_k(u_hbm, i_hbm, ue_hbm, ie_hbm, ub_hbm, ib_hbm, scal_hbm,
                out_hbm,
                uidx, iidx, vu_v, vi_v, bu_v, bi_v, sc_v, scal_v, sem):
        wid = lax.axis_index("s") * _NC + lax.axis_index("c")
        base = wid * _BPW
        pltpu.sync_copy(u_hbm.at[pl.ds(wid * _NCH, _NCH)], uidx)
        pltpu.sync_copy(i_hbm.at[pl.ds(wid * _NCH, _NCH)], iidx)
        pltpu.sync_copy(scal_hbm, scal_v)
        cps = []
        for t in range(_NCH):
            sl = pl.ds(t * _CHUNK, _CHUNK)
            cps.append(pltpu.async_copy(ue_hbm.at[uidx.at[t]], vu_v.at[sl], sem))
            cps.append(pltpu.async_copy(ie_hbm.at[iidx.at[t]], vi_v.at[sl], sem))
            cps.append(pltpu.async_copy(ub_hbm.at[uidx.at[t]], bu_v.at[sl], sem))
            cps.append(pltpu.async_copy(ib_hbm.at[iidx.at[t]], bi_v.at[sl], sem))
        for c in cps:
            c.wait()

        sv = scal_v[...]
        c1 = sv[0]
        c0 = sv[1]
        lane = lax.iota(jnp.int32, _L)

        def group(g, _):
            rows = g * _L + lane
            nu = jnp.zeros((_L,), jnp.float32)
            nv = jnp.zeros((_L,), jnp.float32)
            dot = jnp.zeros((_L,), jnp.float32)
            for d in range(D):
                col = jnp.full((_L,), d, jnp.int32)
                gu = plsc.load_gather(vu_v, [rows, col])
                gi = plsc.load_gather(vi_v, [rows, col])
                nu = nu + gu * gu
                nv = nv + gi * gi
                dot = dot + gu * gi
            ru = _rsqrt(nu + 1e-30)
            rv = _rsqrt(nv + 1e-30)
            lu = nu * ru
            lv = nv * rv
            s1u = 1.0 / (1.0 + lu)
            s1v = 1.0 / (1.0 + lv)
            au = s1u / (1.0 + lu * s1u)
            av = s1v / (1.0 + lv * s1v)
            nuu = nu * au * au
            nvv = nv * av * av
            d2 = nuu + nvv - 2.0 * (au * av) * dot
            den = (1.0 - nuu) * (1.0 - nvv) + 1e-12
            arg = 1.0 + 2.0 * d2 / den
            arg = jnp.maximum(arg, 1.0 + 1e-12)
            t2 = arg * arg - 1.0
            s = _sqrt(t2)
            dist = _log(arg + s)
            gs = pl.ds(g * _L, _L)
            x = bu_v[gs] + bi_v[gs] + dist
            sc_v[gs] = x * c1 + c0
            return ()

        lax.fori_loop(0, _NG, group, (), unroll=False)
        pltpu.sync_copy(sc_v, out_hbm.at[pl.ds(base, _BPW)])

    return fused_k(u2, i2, user_emb, item_emb, ub1, ib1, scal)


def kernel(u, i, user_emb, item_emb, user_bias, item_bias, offset, lin_w, lin_b):
    u2 = u.astype(jnp.int32).reshape(_NW * _NCH, _CHUNK)
    i2 = i.astype(jnp.int32).reshape(_NW * _NCH, _CHUNK)
    w00 = lin_w[0, 0]
    c0 = offset[0] * w00 + lin_b[0]
    scal = jnp.zeros((16,), jnp.float32).at[0].set(w00).at[1].set(c0)
    return _sc_fused(u2, i2, user_emb, item_emb,
                     user_bias[:, 0], item_bias[:, 0], scal)


# speedup vs baseline: 2.1745x; 2.1745x over previous
import functools

import jax
import jax.numpy as jnp
from jax import lax
from jax.experimental import pallas as pl
from jax.experimental.pallas import tpu as pltpu
from jax.experimental.pallas import tpu_sc as plsc

B = 16384
D = 32
_NC = 2
_NS = 16
_NW = _NC * _NS
_BPW = B // _NW
_CHUNK = 128
_NCH = _BPW // _CHUNK
_L = 16
_NG = _BPW // _L
_EB = 4
_NSTEP = _BPW // _EB

_LN2 = 0.6931471805599453
_SQRT2H = 1.4142135


def _rsqrt(x):
    i = plsc.bitcast(x, jnp.int32)
    i = jnp.int32(0x5F3759DF) - (i >> 1)
    y = plsc.bitcast(i, jnp.float32)
    for _ in range(3):
        y = y * (1.5 - 0.5 * x * y * y)
    return y


def _sqrt(x):
    xs = x + 1e-30
    return xs * _rsqrt(xs)


def _log(w):
    iw = plsc.bitcast(w, jnp.int32)
    e = ((iw >> 23) & jnp.int32(0xFF)) - jnp.int32(127)
    m = plsc.bitcast((iw & jnp.int32(0x007FFFFF)) | jnp.int32(0x3F800000),
                     jnp.float32)
    big = m > _SQRT2H
    m = jnp.where(big, m * 0.5, m)
    e = jnp.where(big, e + 1, e)
    z = (m - 1.0) / (m + 1.0)
    z2 = z * z
    p = 2.0 * z * (1.0 + z2 * (1.0 / 3.0 + z2 * (0.2 + z2 * (1.0 / 7.0))))
    return e.astype(jnp.float32) * _LN2 + p


def _sc_fused(u2, i2, u1, i1, ue_t, ie_t, ub1, ib1, scal):
    mesh = plsc.VectorSubcoreMesh(core_axis_name="c", subcore_axis_name="s")

    @functools.partial(
        pl.kernel,
        mesh=mesh,
        compiler_params=pltpu.CompilerParams(
            use_tc_tiling_on_sc=True, needs_layout_passes=False),
        out_type=jax.ShapeDtypeStruct((B,), jnp.float32),
        scratch_types=[
            pltpu.VMEM((_NCH, _CHUNK), jnp.int32),
            pltpu.VMEM((_NCH, _CHUNK), jnp.int32),
            pltpu.VMEM((_BPW + _L,), jnp.int32),
            pltpu.VMEM((_BPW + _L,), jnp.int32),
            pltpu.VMEM((_EB, D, _CHUNK), jnp.float32),
            pltpu.VMEM((_EB, D, _CHUNK), jnp.float32),
            pltpu.VMEM((_BPW // 4, _CHUNK), jnp.float32),
            pltpu.VMEM((_BPW // 4, _CHUNK), jnp.float32),
            pltpu.VMEM((_BPW,), jnp.float32),
            pltpu.VMEM((_BPW,), jnp.float32),
            pltpu.VMEM((_BPW,), jnp.float32),
            pltpu.VMEM((16,), jnp.float32),
            pltpu.SemaphoreType.DMA,
            pltpu.SemaphoreType.DMA,
        ],
    )
    def fused_k(u_hbm, i_hbm, u1_hbm, i1_hbm, ue_hbm, ie_hbm, ub_hbm, ib_hbm,
                scal_hbm, out_hbm,
                uidx, iidx, u1d, i1d, ublk, iblk, vu_v, vi_v,
                bu_v, bi_v, sc_v, scal_v, sem, bsem):
        wid = lax.axis_index("s") * _NC + lax.axis_index("c")
        base = wid * _BPW
        pltpu.sync_copy(u_hbm.at[pl.ds(wid * _NCH, _NCH)], uidx)
        pltpu.sync_copy(i_hbm.at[pl.ds(wid * _NCH, _NCH)], iidx)
        pltpu.sync_copy(u1_hbm.at[pl.ds(base, _BPW + _L)], u1d)
        pltpu.sync_copy(i1_hbm.at[pl.ds(base, _BPW + _L)], i1d)
        pltpu.sync_copy(scal_hbm, scal_v)
        bcps = []
        for t in range(_NCH):
            sl = pl.ds(t * _CHUNK, _CHUNK)
            bcps.append(pltpu.async_copy(ub_hbm.at[uidx.at[t]], bu_v.at[sl], bsem))
            bcps.append(pltpu.async_copy(ib_hbm.at[iidx.at[t]], bi_v.at[sl], bsem))

        lane = lax.iota(jnp.int32, _L)

        def fetch(g, _):
            for b in range(_EB):
                k = g * _EB + b
                uk = u1d[pl.ds(k, _L)][0]
                ik = i1d[pl.ds(k, _L)][0]
                uoff = pl.multiple_of((uk >> 7) << 7, _CHUNK)
                ioff = pl.multiple_of((ik >> 7) << 7, _CHUNK)
                pltpu.async_copy(ue_hbm.at[:, pl.ds(uoff, _CHUNK)],
                                 ublk.at[b], sem)
                pltpu.async_copy(ie_hbm.at[:, pl.ds(ioff, _CHUNK)],
                                 iblk.at[b], sem)
            for b in range(_EB):
                pltpu.make_async_copy(ue_hbm.at[:, pl.ds(0, _CHUNK)],
                                      ublk.at[b], sem).wait()
                pltpu.make_async_copy(ie_hbm.at[:, pl.ds(0, _CHUNK)],
                                      iblk.at[b], sem).wait()
            for b in range(_EB):
                k = g * _EB + b
                uk = u1d[pl.ds(k, _L)][0]
                ik = i1d[pl.ds(k, _L)][0]
                uc = jnp.full((_L,), uk & 127, jnp.int32)
                ic = jnp.full((_L,), ik & 127, jnp.int32)
                kr = jnp.full((_L,), k >> 2, jnp.int32)
                kc = (k & 3) * 32
                for h in range(2):
                    rows = lane + h * _L
                    gu = plsc.load_gather(ublk.at[b], [rows, uc])
                    gi = plsc.load_gather(iblk.at[b], [rows, ic])
                    plsc.store_scatter(vu_v, [kr, kc + rows], gu)
                    plsc.store_scatter(vi_v, [kr, kc + rows], gi)
            return ()

        lax.fori_loop(0, _NSTEP, fetch, (), unroll=False)
        for c in bcps:
            c.wait()

        sv = scal_v[...]
        c1 = sv[0]
        c0 = sv[1]

        def group(g, _):
            e = g * _L + lane
            er = e >> 2
            ec = (e & 3) * 32
            nu = jnp.zeros((_L,), jnp.float32)
            nv = jnp.zeros((_L,), jnp.float32)
            dot = jnp.zeros((_L,), jnp.float32)
            for d in range(D):
                col = ec + d
                gu = plsc.load_gather(vu_v, [er, col])
                gi = plsc.load_gather(vi_v, [er, col])
                nu = nu + gu * gu
                nv = nv + gi * gi
                dot = dot + gu * gi
            ru = _rsqrt(nu + 1e-30)
            rv = _rsqrt(nv + 1e-30)
            lu = nu * ru
            lv = nv * rv
            s1u = 1.0 / (1.0 + lu)
            s1v = 1.0 / (1.0 + lv)
            au = s1u / (1.0 + lu * s1u)
            av = s1v / (1.0 + lv * s1v)
            nuu = nu * au * au
            nvv = nv * av * av
            d2 = nuu + nvv - 2.0 * (au * av) * dot
            den = (1.0 - nuu) * (1.0 - nvv) + 1e-12
            arg = 1.0 + 2.0 * d2 / den
            arg = jnp.maximum(arg, 1.0 + 1e-12)
            s = _sqrt(arg * arg - 1.0)
            dist = _log(arg + s)
            gs = pl.ds(g * _L, _L)
            x = bu_v[gs] + bi_v[gs] + dist
            sc_v[gs] = x * c1 + c0
            return ()

        lax.fori_loop(0, _NG, group, (), unroll=False)
        pltpu.sync_copy(sc_v, out_hbm.at[pl.ds(base, _BPW)])

    return fused_k(u2, i2, u1, i1, ue_t, ie_t, ub1, ib1, scal)


def kernel(u, i, user_emb, item_emb, user_bias, item_bias, offset, lin_w, lin_b):
    u32 = u.astype(jnp.int32)
    i32 = i.astype(jnp.int32)
    u2 = u32.reshape(_NW * _NCH, _CHUNK)
    i2 = i32.reshape(_NW * _NCH, _CHUNK)
    u1 = jnp.concatenate([u32, jnp.zeros((_L,), jnp.int32)])
    i1 = jnp.concatenate([i32, jnp.zeros((_L,), jnp.int32)])
    w00 = lin_w[0, 0]
    c0 = offset[0] * w00 + lin_b[0]
    scal = jnp.zeros((16,), jnp.float32).at[0].set(w00).at[1].set(c0)
    return _sc_fused(u2, i2, u1, i1, user_emb.T, item_emb.T,
                     user_bias[:, 0], item_bias[:, 0], scal)
